# baseline (speedup 1.0000x reference)
import jax
import jax.numpy as jnp
from jax import lax
from jax.experimental import pallas as pl
from jax.experimental.pallas import tpu as pltpu
from jax.experimental.pallas import tpu_sc as plsc

B, T, D = 16, 512, 384
F = 4 * T
L = 16
FRAMES = 1024
G = 64
W = G + 8
NCHUNK = FRAMES // G
IPG = G // L
DV = D // L


def _tile_body(enc_hbm, dur_hbm, out_hbm, dur_v, cnt_v, idx_v, wst_v,
               wins, exps, fsems, wsems):
    wid = lax.axis_index("s") * 2 + lax.axis_index("c")
    b = wid // 2
    f0 = (wid % 2) * FRAMES
    i32 = jnp.int32

    pltpu.sync_copy(dur_hbm.at[b], dur_v)

    for m in range(FRAMES // L):
        cnt_v[pl.ds(m * L, L)] = jnp.zeros((L,), i32)

    one_v = jnp.ones((L,), i32)
    zero_v = jnp.zeros((L,), i32)
    run = i32(0)
    base = i32(0)
    for i in range(T // L):
        v = jnp.maximum(dur_v[pl.ds(i * L, L)], 1)
        ends = plsc.cumsum(v) + run
        k = ends - f0
        plsc.addupdate_scatter(cnt_v, [k], one_v,
                               mask=(k >= 0) & (k < FRAMES))
        base = base + jnp.sum(jnp.where(k < 0, one_v, zero_v))
        run = run + jnp.sum(v)

    row0 = base + b * T
    cap = b * T + (T - 1)

    run = row0
    for c in range(NCHUNK):
        for m in range(IPG):
            v = cnt_v[pl.ds((c * IPG + m) * L, L)]
            s = plsc.cumsum(v) + run
            idx_v[c, pl.ds(m * L, L)] = jnp.minimum(s, cap)
            run = run + jnp.sum(v)

    def _expand(c, slot):
        wst = wst_v[c]

        def _frame(f, carry):
            loc = idx_v[c, pl.ds(f, L)][0] - wst
            for m in range(DV):
                exps[slot][f, pl.ds(m * L, L)] = wins[slot][loc,
                                                            pl.ds(m * L, L)]
            return carry
        lax.fori_loop(0, G, _frame, i32(0))

    def _fetch(c, slot):
        wst = (idx_v[c, pl.ds(0, L)][0] // 8) * 8
        wst = pl.multiple_of(jnp.minimum(wst, B * T - W), 8)
        wst_v[c] = wst
        pltpu.make_async_copy(enc_hbm.at[pl.ds(wst, W)], wins[slot],
                              fsems[slot]).start()

    def _write(c, slot):
        return pltpu.make_async_copy(
            exps[slot], out_hbm.at[b, pl.ds(f0 + c * G, G)], wsems[slot])

    _fetch(0, 0)
    for c in range(NCHUNK):
        s = c % 2
        if c + 1 < NCHUNK:
            _fetch(c + 1, 1 - s)
        pltpu.make_async_copy(enc_hbm.at[pl.ds(i32(0), W)], wins[s],
                              fsems[s]).wait()
        if c >= 2:
            _write(c - 2, s).wait()
        _expand(c, s)
        _write(c, s).start()
    for c in (NCHUNK - 2, NCHUNK - 1):
        _write(c, c % 2).wait()


@jax.jit
def kernel(encoder_output, durations):
    enc_flat = encoder_output.reshape(B * T, D)
    run = pl.kernel(
        _tile_body,
        out_type=jax.ShapeDtypeStruct((B, F, D), jnp.float32),
        mesh=plsc.VectorSubcoreMesh(core_axis_name="c", subcore_axis_name="s"),
        compiler_params=pltpu.CompilerParams(needs_layout_passes=False),
        scratch_types=[
            pltpu.VMEM((T,), jnp.int32),
            pltpu.VMEM((FRAMES,), jnp.int32),
            pltpu.VMEM((NCHUNK, G + L), jnp.int32),
            pltpu.SMEM((NCHUNK,), jnp.int32),
            [pltpu.VMEM((W, D), jnp.float32) for _ in range(2)],
            [pltpu.VMEM((G, D), jnp.float32) for _ in range(2)],
            [pltpu.SemaphoreType.DMA for _ in range(2)],
            [pltpu.SemaphoreType.DMA for _ in range(2)],
        ],
    )
    return run(enc_flat, durations)
